# baseline (speedup 1.0000x reference)
import jax
import jax.numpy as jnp
from jax import lax
from jax.experimental import pallas as pl
from jax.experimental.pallas import tpu as pltpu
from jax.experimental.pallas import tpu_sc as plsc

B = 4096
T = 2048
_ROWS = 128
_NW = 32
_RPW = B // _NW
_L = 16
_NSEG = 24
_SLOTS = _NSEG * _L
_OSL = 512


def _buckets(col, rows, r0):
    key = col >> 7
    jq = lax.broadcasted_iota(jnp.int32, (16, 128), 0)
    oh = (key[None, :] == jq).astype(jnp.float32)
    qr = lax.broadcasted_iota(jnp.int32, (128, 128), 0)
    qc = lax.broadcasted_iota(jnp.int32, (128, 128), 1)
    lt = (qr < qc).astype(jnp.float32)
    ohlt = jnp.einsum("jp,pq->jq", oh, lt,
                      preferred_element_type=jnp.float32)
    rank = jnp.sum(oh * ohlt, axis=0).astype(jnp.int32)
    cnt1 = jnp.sum(oh, axis=1, keepdims=True).astype(jnp.int32)
    nseg1 = (cnt1 + 15) >> 4
    inc = nseg1
    for sh in (1, 2, 4, 8):
        inc = inc + jnp.concatenate(
            [jnp.zeros((sh, 1), jnp.int32), inc[:-sh]], axis=0)
    segb_inc1 = inc
    segb_exc1 = segb_inc1 - nseg1
    segb_exc_b = jnp.broadcast_to(segb_exc1, (16, 128))
    segb_inc_b = jnp.broadcast_to(segb_inc1, (16, 128))
    segbase_q = jnp.sum(oh * segb_exc_b.astype(jnp.float32),
                        axis=0).astype(jnp.int32)
    slot = 16 * segbase_q + rank
    siota = lax.broadcasted_iota(jnp.int32, (_OSL, 128), 0)
    soh = (slot[None, :] == siota).astype(jnp.float32)
    has = jnp.sum(soh, axis=1)
    srows = jnp.sum(soh * rows[None, :].astype(jnp.float32), axis=1)
    slane = jnp.sum(soh * (col & 127)[None, :].astype(jnp.float32), axis=1)
    sdst = jnp.sum(
        soh * lax.broadcasted_iota(jnp.int32, (_OSL, 128), 1)
        .astype(jnp.float32), axis=1)
    s1 = lax.iota(jnp.int32, _OSL)
    pad_rows = r0 + (s1 % 128)
    srows = jnp.where(has > 0, srows.astype(jnp.int32), pad_rows)
    slane = slane.astype(jnp.int32)
    sdst = jnp.where(has > 0, sdst.astype(jnp.int32), -1)
    giota = lax.broadcasted_iota(jnp.int32, (16, 128), 1)
    segcb = jnp.sum((giota >= segb_inc_b).astype(jnp.int32), axis=0)
    segcb = jnp.minimum(segcb, 15)
    j15 = (jq == 15).astype(jnp.int32)
    nsegtot_b = jnp.sum(j15 * segb_inc_b, axis=0)
    i128 = lax.iota(jnp.int32, 128)
    segcb = jnp.where(i128 == 31, nsegtot_b, segcb)
    return srows, slane, sdst, segcb


def _scan_body(x_ref, y_ref, t_ref, xq_ref, yq_ref, tq_ref, tmax_ref,
               nx_ref, ny_ref, nt_ref, fr_ref,
               sri_ref, sli_ref, sdi_ref, sci_ref,
               sr0_ref, sl0_ref, sd0_ref, sc0_ref,
               sr1_ref, sl1_ref, sd1_ref, sc1_ref):
    r0 = pl.program_id(0) * _ROWS
    xb = x_ref[...]
    yb = y_ref[...]
    tb = t_ref[...]
    xq = xq_ref[...][:, None]
    yq = yq_ref[...][:, None]
    tmax = tmax_ref[...]
    iota = lax.broadcasted_iota(jnp.int32, (_ROWS, T), 1)
    cnt_tmax = jnp.sum((tb < tmax[:, None]).astype(jnp.int32), axis=1)
    valid = iota <= cnt_tmax[:, None]
    dx = xb - xq
    dy = yb - yq
    d2 = dx * dx + dy * dy
    d2 = jnp.where(valid, d2, jnp.float32(1e18))
    dmin = jnp.min(d2, axis=1)
    sel = jnp.where(d2 == dmin[:, None], iota, jnp.int32(T))
    idx = jnp.min(sel, axis=1)
    tc = jnp.minimum(jnp.maximum(tq_ref[...], 0.0), tmax)
    cnt_q = jnp.sum((tb < tc[:, None]).astype(jnp.int32), axis=1)
    ti = jnp.clip(cnt_q - 1, 0, T - 2)
    mi = iota == idx[:, None]
    nx_ref[...] = jnp.sum(jnp.where(mi, xb, 0.0), axis=1)
    ny_ref[...] = jnp.sum(jnp.where(mi, yb, 0.0), axis=1)
    nt_ref[...] = jnp.sum(jnp.where(mi, tb, 0.0), axis=1)
    t0 = jnp.sum(jnp.where(iota == ti[:, None], tb, 0.0), axis=1)
    t1 = jnp.sum(jnp.where(iota == (ti + 1)[:, None], tb, 0.0), axis=1)
    frac = (tc - t0) / (t1 - t0 + 1e-12)
    fr_ref[...] = jnp.minimum(jnp.maximum(frac, 0.0), 1.0)
    rows = r0 + lax.iota(jnp.int32, _ROWS)
    for col, (sr, sl, sd, sc) in ((idx, (sri_ref, sli_ref, sdi_ref, sci_ref)),
                                  (ti, (sr0_ref, sl0_ref, sd0_ref, sc0_ref)),
                                  (ti + 1, (sr1_ref, sl1_ref, sd1_ref,
                                            sc1_ref))):
        a, b_, c_, d_ = _buckets(col, rows, r0)
        sr[...] = a
        sl[...] = b_
        sd[...] = c_
        sc[...] = d_


def _scan(ref_x, ref_y, ref_t, x, y, t_query, t_max):
    row_spec = pl.BlockSpec((_ROWS, T), lambda i: (i, 0))
    vec_spec = pl.BlockSpec((_ROWS,), lambda i: (i,))
    seg_spec = pl.BlockSpec((_OSL,), lambda i: (i,))
    scb_spec = pl.BlockSpec((128,), lambda i: (i,))
    seg_ty = jax.ShapeDtypeStruct((_NW * _OSL,), jnp.int32)
    scb_ty = jax.ShapeDtypeStruct((_NW * 128,), jnp.int32)
    vf = jax.ShapeDtypeStruct((B,), jnp.float32)
    return pl.pallas_call(
        _scan_body,
        grid=(B // _ROWS,),
        in_specs=[row_spec, row_spec, row_spec,
                  vec_spec, vec_spec, vec_spec, vec_spec],
        out_specs=[vec_spec] * 4 + [seg_spec, seg_spec, seg_spec, scb_spec] * 3,
        out_shape=[vf] * 4 + [seg_ty, seg_ty, seg_ty, scb_ty] * 3,
    )(ref_x, ref_y, ref_t, x, y, t_query, t_max)


def _sc_phase(tbl_h, rows_v, lane_v, dst_v, cb_v, nst, buf, outbuf, sem):
    i16 = lax.iota(jnp.int32, _L)

    def fire(g, carry):
        cb = cb_v[pl.ds(g, _L)][0]
        pltpu.make_async_copy(
            tbl_h.at[rows_v.at[pl.ds(g * _L, _L)],
                     pl.ds(pl.multiple_of(cb * 128, 128), 128)],
            buf.at[pl.ds(g * _L, _L)], sem).start()
        return carry

    def drain(g, carry):
        pltpu.make_async_copy(
            tbl_h.at[rows_v.at[pl.ds(0, _L)], pl.ds(0, 128)],
            buf.at[pl.ds(0, _L)], sem).wait()
        return carry

    def extract(g, carry):
        lanes = lane_v[pl.ds(g * _L, _L)]
        vals = plsc.load_gather(buf, [g * _L + i16, lanes])
        dst = dst_v[pl.ds(g * _L, _L)]
        plsc.store_scatter(outbuf, [dst], vals, mask=dst >= 0)
        return carry

    lax.fori_loop(0, nst, fire, 0)
    lax.fori_loop(0, nst, drain, 0)
    lax.fori_loop(0, nst, extract, 0)


def _gather_body(th_h, ka_h, vv_h, aa_h, ss_h,
                 sri_h, sli_h, sdi_h, sci_h,
                 sr0_h, sl0_h, sd0_h, sc0_h,
                 sr1_h, sl1_h, sd1_h, sc1_h, fr_h,
                 o_th, o_ka, o_vv, o_aa, o_ss, o_ik, o_iv, o_ia, o_is,
                 rows_v, lane_v, dst_v, cb_v, fr_v, buf,
                 b_th, b_ka, b_vv, b_aa, b_ss,
                 y0k, y0v, y0a, y0s, y1k, y1v, y1a, y1s, sem):
    wid = lax.axis_index("s") * 2 + lax.axis_index("c")
    base = wid * _RPW
    pltpu.sync_copy(fr_h.at[pl.ds(base, _RPW)], fr_v)
    colsets = (
        ((sri_h, sli_h, sdi_h, sci_h),
         ((th_h, b_th), (ka_h, b_ka), (vv_h, b_vv), (aa_h, b_aa),
          (ss_h, b_ss))),
        ((sr0_h, sl0_h, sd0_h, sc0_h),
         ((ka_h, y0k), (vv_h, y0v), (aa_h, y0a), (ss_h, y0s))),
        ((sr1_h, sl1_h, sd1_h, sc1_h),
         ((ka_h, y1k), (vv_h, y1v), (aa_h, y1a), (ss_h, y1s))),
    )
    for (sr_h, sl_h, sd_h, sc_h), tables in colsets:
        pltpu.sync_copy(sr_h.at[pl.ds(wid * _OSL, _SLOTS)], rows_v)
        pltpu.sync_copy(sl_h.at[pl.ds(wid * _OSL, _SLOTS)], lane_v)
        pltpu.sync_copy(sd_h.at[pl.ds(wid * _OSL, _SLOTS)], dst_v)
        pltpu.sync_copy(sc_h.at[pl.ds(wid * 128, 32)], cb_v.at[pl.ds(0, 32)])
        nst = cb_v[pl.ds(16, _L)][15]
        for tbl_h, outbuf in tables:
            _sc_phase(tbl_h, rows_v, lane_v, dst_v, cb_v, nst, buf, outbuf,
                      sem)
    for c in range(_RPW // _L):
        sl = pl.ds(c * _L, _L)
        f = fr_v[sl]
        o_ik_v = y0k[sl] + f * (y1k[sl] - y0k[sl])
        o_iv_v = y0v[sl] + f * (y1v[sl] - y0v[sl])
        o_ia_v = y0a[sl] + f * (y1a[sl] - y0a[sl])
        o_is_v = y0s[sl] + f * (y1s[sl] - y0s[sl])
        y0k[sl] = o_ik_v
        y0v[sl] = o_iv_v
        y0a[sl] = o_ia_v
        y0s[sl] = o_is_v
    for src, out in ((b_th, o_th), (b_ka, o_ka), (b_vv, o_vv), (b_aa, o_aa),
                     (b_ss, o_ss), (y0k, o_ik), (y0v, o_iv), (y0a, o_ia),
                     (y0s, o_is)):
        pltpu.sync_copy(src, out.at[pl.ds(base, _RPW)])


def _gather(th, ka, vv, aa, ss, segs, fr):
    mesh = plsc.VectorSubcoreMesh(core_axis_name="c", subcore_axis_name="s")
    out_type = [jax.ShapeDtypeStruct((B,), jnp.float32) for _ in range(9)]
    scratch = ([pltpu.VMEM((_SLOTS,), jnp.int32) for _ in range(3)]
               + [pltpu.VMEM((48,), jnp.int32),
                  pltpu.VMEM((_RPW,), jnp.float32),
                  pltpu.VMEM((_NSEG * _L, 128), jnp.float32)]
               + [pltpu.VMEM((_RPW,), jnp.float32) for _ in range(13)]
               + [pltpu.SemaphoreType.DMA])
    f = pl.kernel(_gather_body, mesh=mesh, out_type=out_type,
                  scratch_types=scratch,
                  compiler_params=pltpu.CompilerParams(
                      needs_layout_passes=False))
    return f(th, ka, vv, aa, ss, *segs, fr)


_BISECT = False


def kernel(ref_x, ref_y, ref_theta, ref_kappa, ref_v, ref_a, ref_s, ref_t,
           valid_mask, t_max, x, y, t_query):
    if _BISECT:
        seg_i = jnp.zeros((_NW * _OSL,), jnp.int32)
        scb = jnp.zeros((_NW * 128,), jnp.int32)
        segs = [seg_i, seg_i, seg_i, scb] * 3
        fr = jnp.zeros((B,), jnp.float32)
        nx = ny = nt = fr
    else:
        outs = _scan(ref_x, ref_y, ref_t, x, y, t_query, t_max)
        nx, ny, nt, fr = outs[:4]
        segs = outs[4:]
    (o_th, o_ka, o_vv, o_aa, o_ss, o_ik, o_iv, o_ia, o_is) = _gather(
        ref_theta, ref_kappa, ref_v, ref_a, ref_s, segs, fr)
    return jnp.stack([nx, ny, o_th, o_ka, o_vv, o_aa, o_ss, nt,
                      o_ik, o_iv, o_ia, o_is], axis=0)


# speedup vs baseline: 1.5830x; 1.5830x over previous
import jax
import jax.numpy as jnp
from jax import lax
from jax.experimental import pallas as pl
from jax.experimental.pallas import tpu as pltpu
from jax.experimental.pallas import tpu_sc as plsc

B = 4096
T = 2048
_ROWS = 256
_NW = 32
_RPW = B // _NW
_L = 16
_NSEG = 24
_SLOTS = _NSEG * _L
_OSL = 512


def _scan_body(x_ref, y_ref, t_ref, xq_ref, yq_ref, tq_ref, tmax_ref,
               nx_ref, ny_ref, nt_ref, fr_ref, idx_ref, ti_ref):
    xb = x_ref[...]
    yb = y_ref[...]
    tb = t_ref[...]
    xq = xq_ref[...][:, None]
    yq = yq_ref[...][:, None]
    tmax = tmax_ref[...]
    iota = lax.broadcasted_iota(jnp.int32, (_ROWS, T), 1)
    cnt_tmax = jnp.sum((tb < tmax[:, None]).astype(jnp.int32), axis=1)
    valid = iota <= cnt_tmax[:, None]
    dx = xb - xq
    dy = yb - yq
    d2 = dx * dx + dy * dy
    d2 = jnp.where(valid, d2, jnp.float32(1e18))
    dmin = jnp.min(d2, axis=1)
    sel = jnp.where(d2 == dmin[:, None], iota, jnp.int32(T))
    idx = jnp.min(sel, axis=1)
    tc = jnp.minimum(jnp.maximum(tq_ref[...], 0.0), tmax)
    cnt_q = jnp.sum((tb < tc[:, None]).astype(jnp.int32), axis=1)
    ti = jnp.clip(cnt_q - 1, 0, T - 2)
    mi = iota == idx[:, None]
    nx_ref[...] = jnp.sum(jnp.where(mi, xb, 0.0), axis=1)
    ny_ref[...] = jnp.sum(jnp.where(mi, yb, 0.0), axis=1)
    nt_ref[...] = jnp.sum(jnp.where(mi, tb, 0.0), axis=1)
    t0 = jnp.sum(jnp.where(iota == ti[:, None], tb, 0.0), axis=1)
    t1 = jnp.sum(jnp.where(iota == (ti + 1)[:, None], tb, 0.0), axis=1)
    frac = (tc - t0) / (t1 - t0 + 1e-12)
    fr_ref[...] = jnp.minimum(jnp.maximum(frac, 0.0), 1.0)
    idx_ref[...] = idx
    ti_ref[...] = ti


def _scan(ref_x, ref_y, ref_t, x, y, t_query, t_max):
    row_spec = pl.BlockSpec((_ROWS, T), lambda i: (i, 0))
    vec_spec = pl.BlockSpec((_ROWS,), lambda i: (i,))
    vf = jax.ShapeDtypeStruct((B,), jnp.float32)
    vi = jax.ShapeDtypeStruct((B,), jnp.int32)
    return pl.pallas_call(
        _scan_body,
        grid=(B // _ROWS,),
        in_specs=[row_spec, row_spec, row_spec,
                  vec_spec, vec_spec, vec_spec, vec_spec],
        out_specs=[vec_spec] * 6,
        out_shape=[vf, vf, vf, vf, vi, vi],
    )(ref_x, ref_y, ref_t, x, y, t_query, t_max)


def _buckets(key, lane, rows, r0):
    jq = lax.broadcasted_iota(jnp.int32, (16, 128), 0)
    oh = (key[None, :] == jq).astype(jnp.int32)
    pre = oh
    for sh in (1, 2, 4, 8, 16, 32, 64):
        pre = pre + jnp.concatenate(
            [jnp.zeros((16, sh), jnp.int32), pre[:, :128 - sh]], axis=1)
    rank = jnp.sum((pre - oh) * oh, axis=0)
    cnt1 = jnp.sum(oh, axis=1, keepdims=True)
    nseg1 = (cnt1 + 15) >> 4
    inc = nseg1
    for sh in (1, 2, 4, 8):
        inc = inc + jnp.concatenate(
            [jnp.zeros((sh, 1), jnp.int32), inc[:-sh]], axis=0)
    segb_inc_b = jnp.broadcast_to(inc, (16, 128))
    segb_exc_b = jnp.broadcast_to(inc - nseg1, (16, 128))
    segbase_q = jnp.sum(oh * segb_exc_b, axis=0)
    slot = 16 * segbase_q + rank
    siota = lax.broadcasted_iota(jnp.int32, (_OSL, 128), 0)
    soh = (slot[None, :] == siota).astype(jnp.int32)
    qio = lax.broadcasted_iota(jnp.int32, (_OSL, 128), 1)
    has = jnp.sum(soh, axis=1)
    srows = jnp.sum(soh * rows[None, :], axis=1)
    slane = jnp.sum(soh * lane[None, :], axis=1)
    sdst = jnp.sum(soh * qio, axis=1)
    s1 = lax.iota(jnp.int32, _OSL)
    srows = jnp.where(has > 0, srows, r0 + (s1 % 128))
    sdst = jnp.where(has > 0, sdst, -1)
    giota = lax.broadcasted_iota(jnp.int32, (16, 128), 1)
    segcb = jnp.sum((giota >= segb_inc_b).astype(jnp.int32), axis=0)
    segcb = jnp.minimum(segcb, 15)
    nsegtot_b = jnp.sum((jq == 15).astype(jnp.int32) * segb_inc_b, axis=0)
    i128 = lax.iota(jnp.int32, 128)
    segcb = jnp.where(i128 == 31, nsegtot_b, segcb)
    return srows, slane, sdst, segcb


def _bucketize_body(idx_ref, ti_ref,
                    sri_ref, sli_ref, sdi_ref, sci_ref,
                    srw_ref, slw_ref, sdw_ref, scw_ref):
    w0 = pl.program_id(0) * 8
    for w in range(8):
        r0 = (w0 + w) * _RPW
        rows = r0 + lax.iota(jnp.int32, 128)
        idx = idx_ref[w]
        a, b_, c_, d_ = _buckets(idx >> 7, idx & 127, rows, r0)
        sri_ref[pl.ds(w * _OSL, _OSL)] = a
        sli_ref[pl.ds(w * _OSL, _OSL)] = b_
        sdi_ref[pl.ds(w * _OSL, _OSL)] = c_
        sci_ref[pl.ds(w * 128, 128)] = d_
        ti = ti_ref[w]
        wkey = jnp.minimum(ti >> 7, 14)
        a, b_, c_, d_ = _buckets(wkey, ti - 128 * wkey, rows, r0)
        srw_ref[pl.ds(w * _OSL, _OSL)] = a
        slw_ref[pl.ds(w * _OSL, _OSL)] = b_
        sdw_ref[pl.ds(w * _OSL, _OSL)] = c_
        scw_ref[pl.ds(w * 128, 128)] = d_


def _bucketize(idx, ti):
    seg_ty = jax.ShapeDtypeStruct((_NW * _OSL,), jnp.int32)
    scb_ty = jax.ShapeDtypeStruct((_NW * 128,), jnp.int32)
    in_spec = pl.BlockSpec((8, _RPW), lambda i: (i, 0))
    seg_spec = pl.BlockSpec((8 * _OSL,), lambda i: (i,))
    scb_spec = pl.BlockSpec((8 * 128,), lambda i: (i,))
    return pl.pallas_call(
        _bucketize_body,
        grid=(_NW // 8,),
        in_specs=[in_spec, in_spec],
        out_specs=[seg_spec, seg_spec, seg_spec, scb_spec] * 2,
        out_shape=[seg_ty, seg_ty, seg_ty, scb_ty] * 2,
    )(idx.reshape(_NW, _RPW), ti.reshape(_NW, _RPW))


def _sc_phase(tbl_h, rows_v, lane_v, dst_v, cb_v, nst, buf, outbuf, sem,
              width, fr_v):
    i16 = lax.iota(jnp.int32, _L)

    def fire(g, carry):
        cb = cb_v[pl.ds(g, _L)][0]
        pltpu.make_async_copy(
            tbl_h.at[rows_v.at[pl.ds(g * _L, _L)],
                     pl.ds(pl.multiple_of(cb * 128, 128), width)],
            buf.at[pl.ds(g * _L, _L), pl.ds(0, width)], sem).start()
        return carry

    def drain(g, carry):
        pltpu.make_async_copy(
            tbl_h.at[rows_v.at[pl.ds(0, _L)], pl.ds(0, width)],
            buf.at[pl.ds(0, _L), pl.ds(0, width)], sem).wait()
        return carry

    def extract(g, carry):
        lanes = lane_v[pl.ds(g * _L, _L)]
        dst = dst_v[pl.ds(g * _L, _L)]
        vals = plsc.load_gather(buf, [g * _L + i16, lanes])
        if fr_v is not None:
            y1 = plsc.load_gather(buf, [g * _L + i16, lanes + 1])
            fv = plsc.load_gather(fr_v, [jnp.maximum(dst, 0)])
            vals = vals + fv * (y1 - vals)
        plsc.store_scatter(outbuf, [dst], vals, mask=dst >= 0)
        return carry

    lax.fori_loop(0, nst, fire, 0)
    lax.fori_loop(0, nst, drain, 0)
    lax.fori_loop(0, nst, extract, 0)


def _gather_body(th_h, ka_h, vv_h, aa_h, ss_h,
                 sri_h, sli_h, sdi_h, sci_h,
                 srw_h, slw_h, sdw_h, scw_h, fr_h,
                 o_th, o_ka, o_vv, o_aa, o_ss, o_ik, o_iv, o_ia, o_is,
                 rows_v, lane_v, dst_v, cb_v, fr_v, buf,
                 b_th, b_ka, b_vv, b_aa, b_ss, b_ik, b_iv, b_ia, b_is, sem):
    wid = lax.axis_index("s") * 2 + lax.axis_index("c")
    base = wid * _RPW
    pltpu.sync_copy(fr_h.at[pl.ds(base, _RPW)], fr_v)
    colsets = (
        ((sri_h, sli_h, sdi_h, sci_h), 128, None,
         ((th_h, b_th), (ka_h, b_ka), (vv_h, b_vv), (aa_h, b_aa),
          (ss_h, b_ss))),
        ((srw_h, slw_h, sdw_h, scw_h), 256, fr_v,
         ((ka_h, b_ik), (vv_h, b_iv), (aa_h, b_ia), (ss_h, b_is))),
    )
    for (sr_h, sl_h, sd_h, sc_h), width, fr_arg, tables in colsets:
        pltpu.sync_copy(sr_h.at[pl.ds(wid * _OSL, _SLOTS)], rows_v)
        pltpu.sync_copy(sl_h.at[pl.ds(wid * _OSL, _SLOTS)], lane_v)
        pltpu.sync_copy(sd_h.at[pl.ds(wid * _OSL, _SLOTS)], dst_v)
        pltpu.sync_copy(sc_h.at[pl.ds(wid * 128, 32)], cb_v.at[pl.ds(0, 32)])
        nst = cb_v[pl.ds(16, _L)][15]
        for tbl_h, outbuf in tables:
            _sc_phase(tbl_h, rows_v, lane_v, dst_v, cb_v, nst, buf, outbuf,
                      sem, width, fr_arg)
    for src, out in ((b_th, o_th), (b_ka, o_ka), (b_vv, o_vv), (b_aa, o_aa),
                     (b_ss, o_ss), (b_ik, o_ik), (b_iv, o_iv), (b_ia, o_ia),
                     (b_is, o_is)):
        pltpu.sync_copy(src, out.at[pl.ds(base, _RPW)])


def _gather(th, ka, vv, aa, ss, segs, fr):
    mesh = plsc.VectorSubcoreMesh(core_axis_name="c", subcore_axis_name="s")
    out_type = [jax.ShapeDtypeStruct((B,), jnp.float32) for _ in range(9)]
    scratch = ([pltpu.VMEM((_SLOTS,), jnp.int32) for _ in range(3)]
               + [pltpu.VMEM((48,), jnp.int32),
                  pltpu.VMEM((_RPW,), jnp.float32),
                  pltpu.VMEM((_SLOTS, 256), jnp.float32)]
               + [pltpu.VMEM((_RPW,), jnp.float32) for _ in range(9)]
               + [pltpu.SemaphoreType.DMA])
    f = pl.kernel(_gather_body, mesh=mesh, out_type=out_type,
                  scratch_types=scratch,
                  compiler_params=pltpu.CompilerParams(
                      needs_layout_passes=False))
    return f(th, ka, vv, aa, ss, *segs, fr)


def kernel(ref_x, ref_y, ref_theta, ref_kappa, ref_v, ref_a, ref_s, ref_t,
           valid_mask, t_max, x, y, t_query):
    nx, ny, nt, fr, idx, ti = _scan(ref_x, ref_y, ref_t, x, y, t_query, t_max)
    segs = _bucketize(idx, ti)
    (o_th, o_ka, o_vv, o_aa, o_ss, o_ik, o_iv, o_ia, o_is) = _gather(
        ref_theta, ref_kappa, ref_v, ref_a, ref_s, segs, fr)
    return jnp.stack([nx, ny, o_th, o_ka, o_vv, o_aa, o_ss, nt,
                      o_ik, o_iv, o_ia, o_is], axis=0)


# speedup vs baseline: 2.8187x; 1.7806x over previous
import jax
import jax.numpy as jnp
from jax import lax
from jax.experimental import pallas as pl
from jax.experimental.pallas import tpu as pltpu
from jax.experimental.pallas import tpu_sc as plsc

B = 4096
T = 2048
_ROWS = 256
_NW = 32
_RPW = B // _NW
_L = 16
_NSEG = 24
_SLOTS = _NSEG * _L
_OSL = 512


def _scan_body(x_ref, y_ref, t_ref, xq_ref, yq_ref, tq_ref, tmax_ref,
               nx_ref, ny_ref, nt_ref, fr_ref, idx_ref, ti_ref):
    xb = x_ref[...]
    yb = y_ref[...]
    tb = t_ref[...]
    xq = xq_ref[...][:, None]
    yq = yq_ref[...][:, None]
    tmax = tmax_ref[...]
    iota = lax.broadcasted_iota(jnp.int32, (_ROWS, T), 1)
    cnt_tmax = jnp.sum((tb < tmax[:, None]).astype(jnp.int32), axis=1)
    valid = iota <= cnt_tmax[:, None]
    dx = xb - xq
    dy = yb - yq
    d2 = dx * dx + dy * dy
    d2 = jnp.where(valid, d2, jnp.float32(1e18))
    dmin = jnp.min(d2, axis=1)
    sel = jnp.where(d2 == dmin[:, None], iota, jnp.int32(T))
    idx = jnp.min(sel, axis=1)
    tc = jnp.minimum(jnp.maximum(tq_ref[...], 0.0), tmax)
    cnt_q = jnp.sum((tb < tc[:, None]).astype(jnp.int32), axis=1)
    ti = jnp.clip(cnt_q - 1, 0, T - 2)
    mi = iota == idx[:, None]
    nx_ref[...] = jnp.sum(jnp.where(mi, xb, 0.0), axis=1)
    ny_ref[...] = jnp.sum(jnp.where(mi, yb, 0.0), axis=1)
    nt_ref[...] = jnp.sum(jnp.where(mi, tb, 0.0), axis=1)
    t0 = jnp.sum(jnp.where(iota == ti[:, None], tb, 0.0), axis=1)
    t1 = jnp.sum(jnp.where(iota == (ti + 1)[:, None], tb, 0.0), axis=1)
    frac = (tc - t0) / (t1 - t0 + 1e-12)
    fr_ref[...] = jnp.minimum(jnp.maximum(frac, 0.0), 1.0)
    idx_ref[...] = idx
    ti_ref[...] = ti


def _scan(ref_x, ref_y, ref_t, x, y, t_query, t_max):
    row_spec = pl.BlockSpec((_ROWS, T), lambda i: (i, 0))
    vec_spec = pl.BlockSpec((_ROWS,), lambda i: (i,))
    vf = jax.ShapeDtypeStruct((B,), jnp.float32)
    vi = jax.ShapeDtypeStruct((B,), jnp.int32)
    return pl.pallas_call(
        _scan_body,
        grid=(B // _ROWS,),
        in_specs=[row_spec, row_spec, row_spec,
                  vec_spec, vec_spec, vec_spec, vec_spec],
        out_specs=[vec_spec] * 6,
        out_shape=[vf, vf, vf, vf, vi, vi],
    )(ref_x, ref_y, ref_t, x, y, t_query, t_max)


def _buckets(key, lane, rows, r0):
    jq = lax.broadcasted_iota(jnp.int32, (16, 128), 0)
    oh = (key[None, :] == jq).astype(jnp.int32)
    pre = oh
    for sh in (1, 2, 4, 8, 16, 32, 64):
        pre = pre + jnp.concatenate(
            [jnp.zeros((16, sh), jnp.int32), pre[:, :128 - sh]], axis=1)
    rank = jnp.sum((pre - oh) * oh, axis=0)
    cnt1 = jnp.sum(oh, axis=1, keepdims=True)
    nseg1 = (cnt1 + 15) >> 4
    inc = nseg1
    for sh in (1, 2, 4, 8):
        inc = inc + jnp.concatenate(
            [jnp.zeros((sh, 1), jnp.int32), inc[:-sh]], axis=0)
    segb_inc_b = jnp.broadcast_to(inc, (16, 128))
    segb_exc_b = jnp.broadcast_to(inc - nseg1, (16, 128))
    segbase_q = jnp.sum(oh * segb_exc_b, axis=0)
    slot = 16 * segbase_q + rank
    siota = lax.broadcasted_iota(jnp.int32, (_SLOTS, 128), 0)
    soh = (slot[None, :] == siota).astype(jnp.float32)
    kio = lax.broadcasted_iota(jnp.int32, (4, 128), 0)
    qi1 = lax.broadcasted_iota(jnp.int32, (4, 128), 1)
    vals4 = jnp.where(kio == 0, rows[None, :],
                      jnp.where(kio == 1, lane[None, :],
                                jnp.where(kio == 2, qi1, 1))).astype(
                                    jnp.float32)
    m4 = lax.dot_general(vals4, soh, (((1,), (1,)), ((), ())),
                         preferred_element_type=jnp.float32)
    srows = m4[0].astype(jnp.int32)
    slane = m4[1].astype(jnp.int32)
    sdst = m4[2].astype(jnp.int32)
    has = m4[3].astype(jnp.int32)
    s1 = lax.iota(jnp.int32, _SLOTS)
    srows = jnp.where(has > 0, srows, r0 + (s1 % 128))
    sdst = jnp.where(has > 0, sdst, -1)
    giota = lax.broadcasted_iota(jnp.int32, (16, 128), 1)
    segcb = jnp.sum((giota >= segb_inc_b).astype(jnp.int32), axis=0)
    segcb = jnp.minimum(segcb, 15)
    nsegtot_b = jnp.sum((jq == 15).astype(jnp.int32) * segb_inc_b, axis=0)
    i128 = lax.iota(jnp.int32, 128)
    segcb = jnp.where(i128 == 31, nsegtot_b, segcb)
    return srows, slane, sdst, segcb


def _bucketize_body(idx_ref, ti_ref,
                    sri_ref, sli_ref, sdi_ref, sci_ref,
                    srw_ref, slw_ref, sdw_ref, scw_ref):
    w0 = pl.program_id(0) * 8
    for w in range(8):
        r0 = (w0 + w) * _RPW
        rows = r0 + lax.iota(jnp.int32, 128)
        idx = idx_ref[w]
        a, b_, c_, d_ = _buckets(idx >> 7, idx & 127, rows, r0)
        sri_ref[pl.ds(w * _OSL, _SLOTS)] = a
        sli_ref[pl.ds(w * _OSL, _SLOTS)] = b_
        sdi_ref[pl.ds(w * _OSL, _SLOTS)] = c_
        sci_ref[pl.ds(w * 128, 128)] = d_
        ti = ti_ref[w]
        wkey = jnp.minimum(ti >> 7, 14)
        a, b_, c_, d_ = _buckets(wkey, ti - 128 * wkey, rows, r0)
        srw_ref[pl.ds(w * _OSL, _SLOTS)] = a
        slw_ref[pl.ds(w * _OSL, _SLOTS)] = b_
        sdw_ref[pl.ds(w * _OSL, _SLOTS)] = c_
        scw_ref[pl.ds(w * 128, 128)] = d_


def _bucketize(idx, ti):
    seg_ty = jax.ShapeDtypeStruct((_NW * _OSL,), jnp.int32)
    scb_ty = jax.ShapeDtypeStruct((_NW * 128,), jnp.int32)
    in_spec = pl.BlockSpec((8, _RPW), lambda i: (i, 0))
    seg_spec = pl.BlockSpec((8 * _OSL,), lambda i: (i,))
    scb_spec = pl.BlockSpec((8 * 128,), lambda i: (i,))
    return pl.pallas_call(
        _bucketize_body,
        grid=(_NW // 8,),
        in_specs=[in_spec, in_spec],
        out_specs=[seg_spec, seg_spec, seg_spec, scb_spec] * 2,
        out_shape=[seg_ty, seg_ty, seg_ty, scb_ty] * 2,
    )(idx.reshape(_NW, _RPW), ti.reshape(_NW, _RPW))


def _sc_phase(tbl_h, rows_v, lane_v, dst_v, cb_v, nst, buf, outbuf, sem,
              width, fr_v):
    i16 = lax.iota(jnp.int32, _L)

    def fire(g, carry):
        cb = cb_v[pl.ds(g, _L)][0]
        pltpu.make_async_copy(
            tbl_h.at[rows_v.at[pl.ds(g * _L, _L)],
                     pl.ds(pl.multiple_of(cb * 128, 128), width)],
            buf.at[pl.ds(g * _L, _L), pl.ds(0, width)], sem).start()
        return carry

    def drain(g, carry):
        pltpu.make_async_copy(
            tbl_h.at[rows_v.at[pl.ds(0, _L)], pl.ds(0, width)],
            buf.at[pl.ds(0, _L), pl.ds(0, width)], sem).wait()
        return carry

    def extract(g, carry):
        lanes = lane_v[pl.ds(g * _L, _L)]
        dst = dst_v[pl.ds(g * _L, _L)]
        vals = plsc.load_gather(buf, [g * _L + i16, lanes])
        if fr_v is not None:
            y1 = plsc.load_gather(buf, [g * _L + i16, lanes + 1])
            fv = plsc.load_gather(fr_v, [jnp.maximum(dst, 0)])
            vals = vals + fv * (y1 - vals)
        plsc.store_scatter(outbuf, [dst], vals, mask=dst >= 0)
        return carry

    lax.fori_loop(0, nst, fire, 0)
    lax.fori_loop(0, nst, drain, 0)
    lax.fori_loop(0, nst, extract, 0)


def _gather_body(th_h, ka_h, vv_h, aa_h, ss_h,
                 sri_h, sli_h, sdi_h, sci_h,
                 srw_h, slw_h, sdw_h, scw_h, fr_h,
                 o_th, o_ka, o_vv, o_aa, o_ss, o_ik, o_iv, o_ia, o_is,
                 rows_v, lane_v, dst_v, cb_v, fr_v, buf,
                 b_th, b_ka, b_vv, b_aa, b_ss, b_ik, b_iv, b_ia, b_is, sem):
    wid = lax.axis_index("s") * 2 + lax.axis_index("c")
    base = wid * _RPW
    pltpu.sync_copy(fr_h.at[pl.ds(base, _RPW)], fr_v)
    colsets = (
        ((sri_h, sli_h, sdi_h, sci_h), 128, None,
         ((th_h, b_th), (ka_h, b_ka), (vv_h, b_vv), (aa_h, b_aa),
          (ss_h, b_ss))),
        ((srw_h, slw_h, sdw_h, scw_h), 256, fr_v,
         ((ka_h, b_ik), (vv_h, b_iv), (aa_h, b_ia), (ss_h, b_is))),
    )
    for (sr_h, sl_h, sd_h, sc_h), width, fr_arg, tables in colsets:
        pltpu.sync_copy(sr_h.at[pl.ds(wid * _OSL, _SLOTS)], rows_v)
        pltpu.sync_copy(sl_h.at[pl.ds(wid * _OSL, _SLOTS)], lane_v)
        pltpu.sync_copy(sd_h.at[pl.ds(wid * _OSL, _SLOTS)], dst_v)
        pltpu.sync_copy(sc_h.at[pl.ds(wid * 128, 32)], cb_v.at[pl.ds(0, 32)])
        nst = cb_v[pl.ds(16, _L)][15]
        for tbl_h, outbuf in tables:
            _sc_phase(tbl_h, rows_v, lane_v, dst_v, cb_v, nst, buf, outbuf,
                      sem, width, fr_arg)
    for src, out in ((b_th, o_th), (b_ka, o_ka), (b_vv, o_vv), (b_aa, o_aa),
                     (b_ss, o_ss), (b_ik, o_ik), (b_iv, o_iv), (b_ia, o_ia),
                     (b_is, o_is)):
        pltpu.sync_copy(src, out.at[pl.ds(base, _RPW)])


def _gather(th, ka, vv, aa, ss, segs, fr):
    mesh = plsc.VectorSubcoreMesh(core_axis_name="c", subcore_axis_name="s")
    out_type = [jax.ShapeDtypeStruct((B,), jnp.float32) for _ in range(9)]
    scratch = ([pltpu.VMEM((_SLOTS,), jnp.int32) for _ in range(3)]
               + [pltpu.VMEM((48,), jnp.int32),
                  pltpu.VMEM((_RPW,), jnp.float32),
                  pltpu.VMEM((_SLOTS, 256), jnp.float32)]
               + [pltpu.VMEM((_RPW,), jnp.float32) for _ in range(9)]
               + [pltpu.SemaphoreType.DMA])
    f = pl.kernel(_gather_body, mesh=mesh, out_type=out_type,
                  scratch_types=scratch,
                  compiler_params=pltpu.CompilerParams(
                      needs_layout_passes=False))
    return f(th, ka, vv, aa, ss, *segs, fr)


def kernel(ref_x, ref_y, ref_theta, ref_kappa, ref_v, ref_a, ref_s, ref_t,
           valid_mask, t_max, x, y, t_query):
    nx, ny, nt, fr, idx, ti = _scan(ref_x, ref_y, ref_t, x, y, t_query, t_max)
    segs = _bucketize(idx, ti)
    (o_th, o_ka, o_vv, o_aa, o_ss, o_ik, o_iv, o_ia, o_is) = _gather(
        ref_theta, ref_kappa, ref_v, ref_a, ref_s, segs, fr)
    return jnp.stack([nx, ny, o_th, o_ka, o_vv, o_aa, o_ss, nt,
                      o_ik, o_iv, o_ia, o_is], axis=0)
